# baseline (speedup 1.0000x reference)
import jax
import jax.numpy as jnp
from jax import lax
from jax.experimental import pallas as pl
from jax.experimental.pallas import tpu as pltpu

K = 100
C = 1000
B = 1024
RB = 32
GRID = B // RB
NBUF = 4


def _body(idx_ref, L_hbm, P_ref, out_ref, W_ref, acc_ref, lbuf, sems):
    i = pl.program_id(0)

    def start_copy(step):
        buf = lax.rem(step, NBUF)
        pltpu.make_async_copy(
            L_hbm.at[pl.ds(step * RB, RB), :],
            lbuf.at[buf],
            sems.at[buf],
        ).start()

    @pl.when(i == 0)
    def _init():
        P = P_ref[...]
        m = jnp.max(P, axis=1, keepdims=True)
        e = jnp.exp(P - m)
        s = jnp.sum(e, axis=1, keepdims=True)
        W_ref[...] = e / s
        acc_ref[...] = jnp.zeros((8, C), jnp.float32)
        for j in range(NBUF):
            start_copy(jnp.int32(j))

    buf = lax.rem(i, NBUF)
    pltpu.make_async_copy(
        L_hbm.at[pl.ds(i * RB, RB), :], lbuf.at[buf], sems.at[buf],
    ).wait()

    idx = idx_ref[0, 0, :]
    onehot = (idx[:, None]
              == jax.lax.broadcasted_iota(jnp.int32, (RB, K), 1)
              ).astype(jnp.float32)
    g = jnp.dot(onehot, W_ref[...], preferred_element_type=jnp.float32)
    x = lbuf[buf] * g
    n = RB
    while n > 8:
        n //= 2
        x = x[:n, :] + x[n:2 * n, :]
    acc_ref[...] += x

    @pl.when(i + NBUF < GRID)
    def _next():
        start_copy(i + NBUF)

    @pl.when(i == pl.num_programs(0) - 1)
    def _fin():
        out_ref[0, 0] = jnp.sum(acc_ref[...])


@jax.jit
def _run(losses, inputs_idx, params):
    L = losses.reshape(B, C)
    idx3 = inputs_idx.astype(jnp.int32).reshape(GRID, 1, RB)
    out = pl.pallas_call(
        _body,
        grid=(GRID,),
        in_specs=[
            pl.BlockSpec((1, 1, RB), lambda i: (i, 0, 0)),
            pl.BlockSpec(memory_space=pl.ANY),
            pl.BlockSpec((K, C), lambda i: (0, 0)),
        ],
        out_specs=pl.BlockSpec(memory_space=pltpu.SMEM),
        out_shape=jax.ShapeDtypeStruct((1, 1), jnp.float32),
        scratch_shapes=[
            pltpu.VMEM((K, C), jnp.float32),
            pltpu.VMEM((8, C), jnp.float32),
            pltpu.VMEM((NBUF, RB, C), jnp.float32),
            pltpu.SemaphoreType.DMA((NBUF,)),
        ],
    )(idx3, L, params)
    return out[0, 0]


def kernel(losses, inputs_idx, params):
    return _run(losses, inputs_idx, params)


# speedup vs baseline: 2.1402x; 2.1402x over previous
import jax
import jax.numpy as jnp
from jax.experimental import pallas as pl
from jax.experimental.pallas import tpu as pltpu

K = 100
C = 1000
B = 1024
BLK = 256
NBLK = B // BLK


def _body(idx_ref, L_ref, P_ref, out_ref, W_ref, acc_ref):
    i = pl.program_id(0)

    @pl.when(i == 0)
    def _init():
        P = P_ref[...]
        m = jnp.max(P, axis=1, keepdims=True)
        e = jnp.exp(P - m)
        s = jnp.sum(e, axis=1, keepdims=True)
        W_ref[...] = e / s
        acc_ref[...] = jnp.zeros((8, C), jnp.float32)

    idx = idx_ref[pl.ds(i * BLK, BLK)]
    onehot = (idx[:, None]
              == jax.lax.broadcasted_iota(jnp.int32, (BLK, K), 1)
              ).astype(jnp.float32)
    g = jnp.dot(onehot, W_ref[...], preferred_element_type=jnp.float32)
    x = L_ref[...] * g
    n = BLK
    while n > 8:
        n //= 2
        x = x[:n, :] + x[n:2 * n, :]
    acc_ref[...] += x

    @pl.when(i == pl.num_programs(0) - 1)
    def _fin():
        out_ref[0, 0] = jnp.sum(acc_ref[...])


@jax.jit
def _run(losses, inputs_idx, params):
    L = losses.reshape(B, C)
    idx = inputs_idx.astype(jnp.int32)
    out = pl.pallas_call(
        _body,
        grid=(NBLK,),
        in_specs=[
            pl.BlockSpec((B,), lambda i: (0,)),
            pl.BlockSpec((BLK, C), lambda i: (i, 0)),
            pl.BlockSpec((K, C), lambda i: (0, 0)),
        ],
        out_specs=pl.BlockSpec(memory_space=pltpu.SMEM),
        out_shape=jax.ShapeDtypeStruct((1, 1), jnp.float32),
        scratch_shapes=[
            pltpu.VMEM((K, C), jnp.float32),
            pltpu.VMEM((8, C), jnp.float32),
        ],
    )(idx, L, params)
    return out[0, 0]


def kernel(losses, inputs_idx, params):
    return _run(losses, inputs_idx, params)


# speedup vs baseline: 2.3563x; 1.1010x over previous
import jax
import jax.numpy as jnp
from jax.experimental import pallas as pl
from jax.experimental.pallas import tpu as pltpu

K = 100
C = 1000
B = 1024
BLK = 512
NBLK = B // BLK


def _body(idx_ref, L_ref, P_ref, out_ref, W_ref, acc_ref):
    i = pl.program_id(0)

    @pl.when(i == 0)
    def _init():
        P = P_ref[...]
        m = jnp.max(P, axis=1, keepdims=True)
        e = jnp.exp(P - m)
        s = jnp.sum(e, axis=1, keepdims=True)
        W_ref[...] = e / s
        acc_ref[...] = jnp.zeros((8, C), jnp.float32)

    idx = idx_ref[pl.ds(i * BLK, BLK)]
    onehot = (idx[:, None]
              == jax.lax.broadcasted_iota(jnp.int32, (BLK, K), 1)
              ).astype(jnp.float32)
    g = jnp.dot(onehot, W_ref[...], preferred_element_type=jnp.float32)
    x = L_ref[...] * g
    n = BLK
    while n > 8:
        n //= 2
        x = x[:n, :] + x[n:2 * n, :]
    acc_ref[...] += x

    @pl.when(i == pl.num_programs(0) - 1)
    def _fin():
        out_ref[0, 0] = jnp.sum(acc_ref[...])


@jax.jit
def _run(losses, inputs_idx, params):
    L = losses.reshape(B, C)
    idx = inputs_idx.astype(jnp.int32)
    out = pl.pallas_call(
        _body,
        grid=(NBLK,),
        in_specs=[
            pl.BlockSpec((B,), lambda i: (0,)),
            pl.BlockSpec((BLK, C), lambda i: (i, 0)),
            pl.BlockSpec((K, C), lambda i: (0, 0)),
        ],
        out_specs=pl.BlockSpec(memory_space=pltpu.SMEM),
        out_shape=jax.ShapeDtypeStruct((1, 1), jnp.float32),
        scratch_shapes=[
            pltpu.VMEM((K, C), jnp.float32),
            pltpu.VMEM((8, C), jnp.float32),
        ],
    )(idx, L, params)
    return out[0, 0]


def kernel(losses, inputs_idx, params):
    return _run(losses, inputs_idx, params)
